# baseline (speedup 1.0000x reference)
import functools

import jax
import jax.numpy as jnp
from jax import lax
from jax.experimental import pallas as pl
from jax.experimental.pallas import tpu as pltpu
from jax.experimental.pallas import tpu_sc as plsc

N = 10000
D = 128
NG = 64
E = 320000
K = 3

NC = 2
NS = 16
NWK = NC * NS
BPW = 80
E2 = NWK * BPW * 128
NP = 10240
RPS = NP // NS
BLK = 2000
GRID = N // BLK
WIN = 16

_mesh = lambda: plsc.VectorSubcoreMesh(core_axis_name="c", subcore_axis_name="s")


def _sc_prop(z, src_g, dst_g):
    @functools.partial(
        pl.kernel,
        mesh=_mesh(),
        out_type=jax.ShapeDtypeStruct((NC, NP, D), jnp.float32),
        scratch_types=[
            pltpu.VMEM((WIN, 128), jnp.int32),
            pltpu.VMEM((WIN, 128), jnp.int32),
            pltpu.VMEM((128, D), jnp.float32),
            pltpu.VMEM((128, D), jnp.float32),
            pltpu.VMEM_SHARED((NP, D), jnp.float32),
            pltpu.SemaphoreType.DMA,
            pltpu.SemaphoreType.DMA,
            pltpu.SemaphoreType.DMA,
            pltpu.SemaphoreType.DMA,
        ],
    )
    def prop(z_hbm, src_hbm, dst_hbm, out_hbm, src_v, dst_v, b0, b1,
             acc, g0, g1, s0, s1):
        c = lax.axis_index("c")
        s = lax.axis_index("s")
        wid = s * NC + c

        def zrow(i, carry):
            for j in range(D // 16):
                b0[i, pl.ds(j * 16, 16)] = jnp.zeros((16,), jnp.float32)
            return carry

        lax.fori_loop(0, 128, zrow, 0)
        rbase = s * RPS
        for k in range(RPS // 128):
            pltpu.sync_copy(b0, acc.at[pl.ds(rbase + k * 128, 128)])
        plsc.subcore_barrier()

        ib = wid * BPW
        for w in range(BPW // WIN):
            pltpu.sync_copy(src_hbm.at[pl.ds(ib + w * WIN, WIN)], src_v)
            pltpu.sync_copy(dst_hbm.at[pl.ds(ib + w * WIN, WIN)], dst_v)
            pltpu.async_copy(z_hbm.at[src_v.at[0]], b0, g0)
            pltpu.async_copy(z_hbm.at[src_v.at[1]], b1, g1)

            def body(p, carry):
                jA = 2 * p
                jB = 2 * p + 1
                pltpu.make_async_copy(z_hbm.at[src_v.at[jA]], b0, g0).wait()
                pltpu.async_copy(b0, acc.at[dst_v.at[jA]], s0, add=True)
                pltpu.make_async_copy(z_hbm.at[src_v.at[jB]], b1, g1).wait()
                pltpu.async_copy(b1, acc.at[dst_v.at[jB]], s1, add=True)
                jC = jnp.minimum(jA + 2, WIN - 1)
                jD = jnp.minimum(jB + 2, WIN - 1)
                pltpu.make_async_copy(b0, acc.at[dst_v.at[jA]], s0).wait()
                pltpu.async_copy(z_hbm.at[src_v.at[jC]], b0, g0)
                pltpu.make_async_copy(b1, acc.at[dst_v.at[jB]], s1).wait()
                pltpu.async_copy(z_hbm.at[src_v.at[jD]], b1, g1)
                return carry

            lax.fori_loop(0, WIN // 2, body, 0)
            pltpu.make_async_copy(z_hbm.at[src_v.at[WIN - 1]], b0, g0).wait()
            pltpu.make_async_copy(z_hbm.at[src_v.at[WIN - 1]], b1, g1).wait()
        plsc.subcore_barrier()
        pltpu.sync_copy(acc.at[pl.ds(rbase, RPS)],
                        out_hbm.at[c, pl.ds(rbase, RPS)])

    return prop(z, src_g, dst_g)


def _row_spec(last):
    return pl.BlockSpec((BLK, last), lambda i: (i, 0))


def _core_spec(core, last):
    return pl.BlockSpec((1, BLK, last), lambda i, _c=core: (_c, i, 0))


def _full_spec(a, b):
    return pl.BlockSpec((a, b), lambda i: (0, 0))


def _k1_body(dga, dgb, xr, dfr, bsr, h0_ref, z0_ref, dis_ref):
    deg = dga[0, :, 0:1] + dgb[0, :, 0:1]
    dis = jnp.where(deg > 0.0, lax.rsqrt(jnp.maximum(deg, 1.0)), 0.0)
    h0 = jnp.dot(xr[...], dfr[...], preferred_element_type=jnp.float32) + bsr[...]
    disb = dis * jnp.ones((1, D), jnp.float32)
    h0_ref[...] = h0
    dis_ref[...] = disb
    z0_ref[...] = disb * h0


def _tc_k1(degp, x16, diff16, basev):
    return pl.pallas_call(
        _k1_body,
        grid=(GRID,),
        in_specs=[
            _core_spec(0, D), _core_spec(1, D),
            _row_spec(16), _full_spec(16, D), _full_spec(1, D),
        ],
        out_specs=[_row_spec(D)] * 3,
        out_shape=[jax.ShapeDtypeStruct((N, D), jnp.float32)] * 3,
    )(degp, degp, x16, diff16, basev)


def _k2_body(gpa, gpb, hr, dr, w0r, w1r, acc_ref, u_ref):
    g = gpa[0] + gpb[0]
    d_ = dr[...]
    tx1 = -(d_ * g)
    acc_ref[...] = (jnp.dot(hr[...], w0r[...], preferred_element_type=jnp.float32)
                    + jnp.dot(tx1, w1r[...], preferred_element_type=jnp.float32))
    u_ref[...] = d_ * tx1


def _tc_k2(gp, h, disb, w0, w1):
    return pl.pallas_call(
        _k2_body,
        grid=(GRID,),
        in_specs=[
            _core_spec(0, D), _core_spec(1, D),
            _row_spec(D), _row_spec(D), _full_spec(D, D), _full_spec(D, D),
        ],
        out_specs=[_row_spec(D)] * 2,
        out_shape=[jax.ShapeDtypeStruct((N, D), jnp.float32)] * 2,
    )(gp, gp, h, disb, w0, w1)


def _k3_body(gpa, gpb, accr, hr, dr, w2r, br, hn_ref, zn_ref):
    g = gpa[0] + gpb[0]
    d_ = dr[...]
    tx2 = -2.0 * (d_ * g) - hr[...]
    hn = jnp.maximum(
        accr[...] + jnp.dot(tx2, w2r[...], preferred_element_type=jnp.float32)
        + br[...], 0.0)
    hn_ref[...] = hn
    zn_ref[...] = d_ * hn


def _tc_k3(gp, acc, h, disb, w2, b):
    return pl.pallas_call(
        _k3_body,
        grid=(GRID,),
        in_specs=[
            _core_spec(0, D), _core_spec(1, D),
            _row_spec(D), _row_spec(D), _row_spec(D),
            _full_spec(D, D), _full_spec(1, D),
        ],
        out_specs=[_row_spec(D)] * 2,
        out_shape=[jax.ShapeDtypeStruct((N, D), jnp.float32)] * 2,
    )(gp, gp, acc, h, disb, w2, b)


def _k5_body(gpa, gpb, accr, hr, dr, w2r, br, hn_ref, st_ref, sacc):
    i = pl.program_id(0)
    g = gpa[0] + gpb[0]
    d_ = dr[...]
    tx2 = -2.0 * (d_ * g) - hr[...]
    hn = jnp.maximum(
        accr[...] + jnp.dot(tx2, w2r[...], preferred_element_type=jnp.float32)
        + br[...], 0.0)
    hn_ref[...] = hn
    ps = jnp.concatenate(
        [jnp.sum(hn, axis=0, keepdims=True),
         jnp.sum(hn * hn, axis=0, keepdims=True)], axis=0)

    @pl.when(i == 0)
    def _():
        sacc[...] = ps

    @pl.when(i > 0)
    def _():
        sacc[...] = sacc[...] + ps

    @pl.when(i == GRID - 1)
    def _():
        st_ref[...] = sacc[...]


def _tc_k5(gp, acc, h, disb, w2, b):
    return pl.pallas_call(
        _k5_body,
        grid=(GRID,),
        in_specs=[
            _core_spec(0, D), _core_spec(1, D),
            _row_spec(D), _row_spec(D), _row_spec(D),
            _full_spec(D, D), _full_spec(1, D),
        ],
        out_specs=[_row_spec(D), _full_spec(2, D)],
        out_shape=[jax.ShapeDtypeStruct((N, D), jnp.float32),
                   jax.ShapeDtypeStruct((2, D), jnp.float32)],
        scratch_shapes=[pltpu.VMEM((2, D), jnp.float32)],
    )(gp, gp, acc, h, disb, w2, b)


def _k6_body(hr, btr, str_, gmr, btr2, w1r, b1r, w2r, b2r, out_ref, pool, cnt):
    i = pl.program_id(0)

    @pl.when(i == 0)
    def _():
        pool[...] = jnp.zeros((NG, D), jnp.float32)
        cnt[...] = jnp.zeros((NG, 1), jnp.float32)

    mask = (btr[...] == lax.broadcasted_iota(jnp.int32, (1, NG), 1)
            ).astype(jnp.float32)
    pool[...] = pool[...] + lax.dot_general(
        mask, hr[...], (((0,), (0,)), ((), ())),
        preferred_element_type=jnp.float32)
    cnt[...] = cnt[...] + lax.dot_general(
        mask, jnp.ones((BLK, 1), jnp.float32), (((0,), (0,)), ((), ())),
        preferred_element_type=jnp.float32)

    @pl.when(i == GRID - 1)
    def _():
        mean = str_[0:1, :] * (1.0 / N)
        var = str_[1:2, :] * (1.0 / N) - mean * mean
        inv = lax.rsqrt(var + 1e-5)
        pm = pool[...] / jnp.maximum(cnt[...], 1.0)
        y = (pm - mean) * inv * gmr[...] + btr2[...]
        r = jnp.maximum(
            jnp.dot(y, w1r[...], preferred_element_type=jnp.float32)
            + b1r[...], 0.0)
        out_ref[...] = (jnp.dot(r, w2r[...], preferred_element_type=jnp.float32)
                        + b2r[...])


def _tc_k6(h2, bat2, stats, gam, bet, w1, b1, w2, b2):
    return pl.pallas_call(
        _k6_body,
        grid=(GRID,),
        in_specs=[
            _row_spec(D), _row_spec(1), _full_spec(2, D),
            _full_spec(1, D), _full_spec(1, D),
            _full_spec(D, 16), _full_spec(1, 16),
            _full_spec(16, 2), _full_spec(1, 2),
        ],
        out_specs=pl.BlockSpec((NG, 2), lambda i: (0, 0)),
        out_shape=jax.ShapeDtypeStruct((NG, 2), jnp.float32),
        scratch_shapes=[pltpu.VMEM((NG, D), jnp.float32),
                        pltpu.VMEM((NG, 1), jnp.float32)],
    )(h2, bat2, stats, gam, bet, w1, b1, w2, b2)


def kernel(x, edge_index, batch, atom_embs, conv1_W, conv1_b, conv3_W, conv3_b,
           bn_gamma, bn_beta, lin1_W, lin1_b, lin2_W, lin2_b):
    f32 = jnp.float32
    x16 = jnp.pad(x.astype(f32), ((0, 0), (0, 7)))
    diff16 = jnp.pad(
        jnp.stack([atom_embs[i][1] - atom_embs[i][0] for i in range(9)]),
        ((0, 7), (0, 0)))
    basev = functools.reduce(
        lambda a, b: a + b, [atom_embs[i][0] for i in range(9)]).reshape(1, D)

    src = edge_index[0]
    dst = edge_index[1]
    pad = E2 - E
    pidx = jnp.arange(pad, dtype=jnp.int32)
    dump = N + pidx % (NP - N)
    zpad = pidx % 128
    src_g = jnp.concatenate([src, zpad]).reshape(E2 // 128, 128)
    src_d = jnp.concatenate([src, dump]).reshape(E2 // 128, 128)
    dst_g = jnp.concatenate([dst, dump]).reshape(E2 // 128, 128)
    bat2 = batch.reshape(N, 1)

    degp = _sc_prop(jnp.ones((N, D), f32), src_g, src_d)
    h0, z0, disb = _tc_k1(degp, x16, diff16, basev)

    g1 = _sc_prop(z0, src_g, dst_g)
    acc1, u1 = _tc_k2(g1, h0, disb, conv1_W[0], conv1_W[1])
    g2 = _sc_prop(u1, src_g, dst_g)
    h1, z1 = _tc_k3(g2, acc1, h0, disb, conv1_W[2], conv1_b.reshape(1, D))

    g3 = _sc_prop(z1, src_g, dst_g)
    acc2, u2 = _tc_k2(g3, h1, disb, conv3_W[0], conv3_W[1])
    g4 = _sc_prop(u2, src_g, dst_g)
    h2, stats = _tc_k5(g4, acc2, h1, disb, conv3_W[2], conv3_b.reshape(1, D))

    return _tc_k6(h2, bat2, stats, bn_gamma.reshape(1, D),
                  bn_beta.reshape(1, D), lin1_W, lin1_b.reshape(1, 16),
                  lin2_W, lin2_b.reshape(1, 2))


# speedup vs baseline: 1.1271x; 1.1271x over previous
import functools

import jax
import jax.numpy as jnp
from jax import lax
from jax.experimental import pallas as pl
from jax.experimental.pallas import tpu as pltpu
from jax.experimental.pallas import tpu_sc as plsc

N = 10000
D = 128
NG = 64
E = 320000
K = 3

NC = 2
NS = 16
NWK = NC * NS
BPW = 80
E2 = NWK * BPW * 128
NP = 10240
RPS = NP // NS
BLK = 2000
GRID = N // BLK
WIN = 16

_mesh = lambda: plsc.VectorSubcoreMesh(core_axis_name="c", subcore_axis_name="s")


def _sc_prop(z, src_g, dst_g):
    @functools.partial(
        pl.kernel,
        mesh=_mesh(),
        out_type=jax.ShapeDtypeStruct((NC, NP, D), jnp.float32),
        scratch_types=[
            pltpu.VMEM((WIN, 128), jnp.int32),
            pltpu.VMEM((WIN, 128), jnp.int32),
            pltpu.VMEM((128, D), jnp.float32),
            pltpu.VMEM((128, D), jnp.float32),
            pltpu.VMEM_SHARED((NP, D), jnp.float32),
            pltpu.SemaphoreType.DMA,
            pltpu.SemaphoreType.DMA,
            pltpu.SemaphoreType.DMA,
            pltpu.SemaphoreType.DMA,
        ],
    )
    def prop(z_hbm, src_hbm, dst_hbm, out_hbm, src_v, dst_v, b0, b1,
             acc, g0, g1, s0, s1):
        c = lax.axis_index("c")
        s = lax.axis_index("s")
        wid = s * NC + c

        def zrow(i, carry):
            for j in range(D // 16):
                b0[i, pl.ds(j * 16, 16)] = jnp.zeros((16,), jnp.float32)
            return carry

        lax.fori_loop(0, 128, zrow, 0)
        rbase = s * RPS
        for k in range(RPS // 128):
            pltpu.sync_copy(b0, acc.at[pl.ds(rbase + k * 128, 128)])
        plsc.subcore_barrier()

        ib = wid * BPW
        for w in range(BPW // WIN):
            pltpu.sync_copy(src_hbm.at[pl.ds(ib + w * WIN, WIN)], src_v)
            pltpu.sync_copy(dst_hbm.at[pl.ds(ib + w * WIN, WIN)], dst_v)
            pltpu.async_copy(z_hbm.at[src_v.at[0]], b0, g0)

            def body(p, carry):
                jA = 2 * p
                jB = 2 * p + 1
                pltpu.make_async_copy(z_hbm.at[src_v.at[jA]], b0, g0).wait()
                pltpu.async_copy(z_hbm.at[src_v.at[jB]], b1, g1)
                pltpu.sync_copy(b0, acc.at[dst_v.at[jA]], add=True)
                pltpu.make_async_copy(z_hbm.at[src_v.at[jB]], b1, g1).wait()
                jC = jnp.minimum(jB + 1, WIN - 1)
                pltpu.async_copy(z_hbm.at[src_v.at[jC]], b0, g0)
                pltpu.sync_copy(b1, acc.at[dst_v.at[jB]], add=True)
                return carry

            lax.fori_loop(0, WIN // 2, body, 0)
            pltpu.make_async_copy(z_hbm.at[src_v.at[WIN - 1]], b0, g0).wait()
        plsc.subcore_barrier()
        pltpu.sync_copy(acc.at[pl.ds(rbase, RPS)],
                        out_hbm.at[c, pl.ds(rbase, RPS)])

    return prop(z, src_g, dst_g)


def _row_spec(last):
    return pl.BlockSpec((BLK, last), lambda i: (i, 0))


def _core_spec(core, last):
    return pl.BlockSpec((1, BLK, last), lambda i, _c=core: (_c, i, 0))


def _full_spec(a, b):
    return pl.BlockSpec((a, b), lambda i: (0, 0))


def _k1_body(dga, dgb, xr, dfr, bsr, h0_ref, z0_ref, dis_ref):
    deg = dga[0, :, 0:1] + dgb[0, :, 0:1]
    dis = jnp.where(deg > 0.0, lax.rsqrt(jnp.maximum(deg, 1.0)), 0.0)
    h0 = jnp.dot(xr[...], dfr[...], preferred_element_type=jnp.float32) + bsr[...]
    disb = dis * jnp.ones((1, D), jnp.float32)
    h0_ref[...] = h0
    dis_ref[...] = disb
    z0_ref[...] = disb * h0


def _tc_k1(degp, x16, diff16, basev):
    return pl.pallas_call(
        _k1_body,
        grid=(GRID,),
        in_specs=[
            _core_spec(0, D), _core_spec(1, D),
            _row_spec(16), _full_spec(16, D), _full_spec(1, D),
        ],
        out_specs=[_row_spec(D)] * 3,
        out_shape=[jax.ShapeDtypeStruct((N, D), jnp.float32)] * 3,
    )(degp, degp, x16, diff16, basev)


def _k2_body(gpa, gpb, hr, dr, w0r, w1r, acc_ref, u_ref):
    g = gpa[0] + gpb[0]
    d_ = dr[...]
    tx1 = -(d_ * g)
    acc_ref[...] = (jnp.dot(hr[...], w0r[...], preferred_element_type=jnp.float32)
                    + jnp.dot(tx1, w1r[...], preferred_element_type=jnp.float32))
    u_ref[...] = d_ * tx1


def _tc_k2(gp, h, disb, w0, w1):
    return pl.pallas_call(
        _k2_body,
        grid=(GRID,),
        in_specs=[
            _core_spec(0, D), _core_spec(1, D),
            _row_spec(D), _row_spec(D), _full_spec(D, D), _full_spec(D, D),
        ],
        out_specs=[_row_spec(D)] * 2,
        out_shape=[jax.ShapeDtypeStruct((N, D), jnp.float32)] * 2,
    )(gp, gp, h, disb, w0, w1)


def _k3_body(gpa, gpb, accr, hr, dr, w2r, br, hn_ref, zn_ref):
    g = gpa[0] + gpb[0]
    d_ = dr[...]
    tx2 = -2.0 * (d_ * g) - hr[...]
    hn = jnp.maximum(
        accr[...] + jnp.dot(tx2, w2r[...], preferred_element_type=jnp.float32)
        + br[...], 0.0)
    hn_ref[...] = hn
    zn_ref[...] = d_ * hn


def _tc_k3(gp, acc, h, disb, w2, b):
    return pl.pallas_call(
        _k3_body,
        grid=(GRID,),
        in_specs=[
            _core_spec(0, D), _core_spec(1, D),
            _row_spec(D), _row_spec(D), _row_spec(D),
            _full_spec(D, D), _full_spec(1, D),
        ],
        out_specs=[_row_spec(D)] * 2,
        out_shape=[jax.ShapeDtypeStruct((N, D), jnp.float32)] * 2,
    )(gp, gp, acc, h, disb, w2, b)


def _k5_body(gpa, gpb, accr, hr, dr, w2r, br, hn_ref, st_ref, sacc):
    i = pl.program_id(0)
    g = gpa[0] + gpb[0]
    d_ = dr[...]
    tx2 = -2.0 * (d_ * g) - hr[...]
    hn = jnp.maximum(
        accr[...] + jnp.dot(tx2, w2r[...], preferred_element_type=jnp.float32)
        + br[...], 0.0)
    hn_ref[...] = hn
    ps = jnp.concatenate(
        [jnp.sum(hn, axis=0, keepdims=True),
         jnp.sum(hn * hn, axis=0, keepdims=True)], axis=0)

    @pl.when(i == 0)
    def _():
        sacc[...] = ps

    @pl.when(i > 0)
    def _():
        sacc[...] = sacc[...] + ps

    @pl.when(i == GRID - 1)
    def _():
        st_ref[...] = sacc[...]


def _tc_k5(gp, acc, h, disb, w2, b):
    return pl.pallas_call(
        _k5_body,
        grid=(GRID,),
        in_specs=[
            _core_spec(0, D), _core_spec(1, D),
            _row_spec(D), _row_spec(D), _row_spec(D),
            _full_spec(D, D), _full_spec(1, D),
        ],
        out_specs=[_row_spec(D), _full_spec(2, D)],
        out_shape=[jax.ShapeDtypeStruct((N, D), jnp.float32),
                   jax.ShapeDtypeStruct((2, D), jnp.float32)],
        scratch_shapes=[pltpu.VMEM((2, D), jnp.float32)],
    )(gp, gp, acc, h, disb, w2, b)


def _k6_body(hr, btr, str_, gmr, btr2, w1r, b1r, w2r, b2r, out_ref, pool, cnt):
    i = pl.program_id(0)

    @pl.when(i == 0)
    def _():
        pool[...] = jnp.zeros((NG, D), jnp.float32)
        cnt[...] = jnp.zeros((NG, 1), jnp.float32)

    mask = (btr[...] == lax.broadcasted_iota(jnp.int32, (1, NG), 1)
            ).astype(jnp.float32)
    pool[...] = pool[...] + lax.dot_general(
        mask, hr[...], (((0,), (0,)), ((), ())),
        preferred_element_type=jnp.float32)
    cnt[...] = cnt[...] + lax.dot_general(
        mask, jnp.ones((BLK, 1), jnp.float32), (((0,), (0,)), ((), ())),
        preferred_element_type=jnp.float32)

    @pl.when(i == GRID - 1)
    def _():
        mean = str_[0:1, :] * (1.0 / N)
        var = str_[1:2, :] * (1.0 / N) - mean * mean
        inv = lax.rsqrt(var + 1e-5)
        pm = pool[...] / jnp.maximum(cnt[...], 1.0)
        y = (pm - mean) * inv * gmr[...] + btr2[...]
        r = jnp.maximum(
            jnp.dot(y, w1r[...], preferred_element_type=jnp.float32)
            + b1r[...], 0.0)
        out_ref[...] = (jnp.dot(r, w2r[...], preferred_element_type=jnp.float32)
                        + b2r[...])


def _tc_k6(h2, bat2, stats, gam, bet, w1, b1, w2, b2):
    return pl.pallas_call(
        _k6_body,
        grid=(GRID,),
        in_specs=[
            _row_spec(D), _row_spec(1), _full_spec(2, D),
            _full_spec(1, D), _full_spec(1, D),
            _full_spec(D, 16), _full_spec(1, 16),
            _full_spec(16, 2), _full_spec(1, 2),
        ],
        out_specs=pl.BlockSpec((NG, 2), lambda i: (0, 0)),
        out_shape=jax.ShapeDtypeStruct((NG, 2), jnp.float32),
        scratch_shapes=[pltpu.VMEM((NG, D), jnp.float32),
                        pltpu.VMEM((NG, 1), jnp.float32)],
    )(h2, bat2, stats, gam, bet, w1, b1, w2, b2)


def kernel(x, edge_index, batch, atom_embs, conv1_W, conv1_b, conv3_W, conv3_b,
           bn_gamma, bn_beta, lin1_W, lin1_b, lin2_W, lin2_b):
    f32 = jnp.float32
    x16 = jnp.pad(x.astype(f32), ((0, 0), (0, 7)))
    diff16 = jnp.pad(
        jnp.stack([atom_embs[i][1] - atom_embs[i][0] for i in range(9)]),
        ((0, 7), (0, 0)))
    basev = functools.reduce(
        lambda a, b: a + b, [atom_embs[i][0] for i in range(9)]).reshape(1, D)

    src = edge_index[0]
    dst = edge_index[1]
    pad = E2 - E
    pidx = jnp.arange(pad, dtype=jnp.int32)
    dump = N + pidx % (NP - N)
    zpad = pidx % 128
    src_g = jnp.concatenate([src, zpad]).reshape(E2 // 128, 128)
    src_d = jnp.concatenate([src, dump]).reshape(E2 // 128, 128)
    dst_g = jnp.concatenate([dst, dump]).reshape(E2 // 128, 128)
    bat2 = batch.reshape(N, 1)

    degp = _sc_prop(jnp.ones((N, D), f32), src_g, src_d)
    h0, z0, disb = _tc_k1(degp, x16, diff16, basev)

    g1 = _sc_prop(z0, src_g, dst_g)
    acc1, u1 = _tc_k2(g1, h0, disb, conv1_W[0], conv1_W[1])
    g2 = _sc_prop(u1, src_g, dst_g)
    h1, z1 = _tc_k3(g2, acc1, h0, disb, conv1_W[2], conv1_b.reshape(1, D))

    g3 = _sc_prop(z1, src_g, dst_g)
    acc2, u2 = _tc_k2(g3, h1, disb, conv3_W[0], conv3_W[1])
    g4 = _sc_prop(u2, src_g, dst_g)
    h2, stats = _tc_k5(g4, acc2, h1, disb, conv3_W[2], conv3_b.reshape(1, D))

    return _tc_k6(h2, bat2, stats, bn_gamma.reshape(1, D),
                  bn_beta.reshape(1, D), lin1_W, lin1_b.reshape(1, 16),
                  lin2_W, lin2_b.reshape(1, 2))


# speedup vs baseline: 1.1371x; 1.0089x over previous
import functools

import jax
import jax.numpy as jnp
from jax import lax
from jax.experimental import pallas as pl
from jax.experimental.pallas import tpu as pltpu
from jax.experimental.pallas import tpu_sc as plsc

N = 10000
D = 128
NG = 64
E = 320000
K = 3

NC = 2
NS = 16
NWK = NC * NS
BPW = 80
E2 = NWK * BPW * 128
NP = 10240
RPS = NP // NS
BLK = 2000
GRID = N // BLK
WIN = 16

_mesh = lambda: plsc.VectorSubcoreMesh(core_axis_name="c", subcore_axis_name="s")


def _sc_prop(z, src_g, dst_g):
    @functools.partial(
        pl.kernel,
        mesh=_mesh(),
        out_type=jax.ShapeDtypeStruct((NC, NP, D), jnp.float32),
        scratch_types=[
            pltpu.VMEM((WIN, 128), jnp.int32),
            pltpu.VMEM((WIN, 128), jnp.int32),
            pltpu.VMEM((128, D), jnp.float32),
            pltpu.VMEM((128, D), jnp.float32),
            pltpu.VMEM_SHARED((NP, D), jnp.float32),
            pltpu.SemaphoreType.DMA,
            pltpu.SemaphoreType.DMA,
            pltpu.SemaphoreType.DMA,
            pltpu.SemaphoreType.DMA,
        ],
    )
    def prop(z_hbm, src_hbm, dst_hbm, out_hbm, src_v, dst_v, b0, b1,
             acc, g0, g1, s0, s1):
        c = lax.axis_index("c")
        s = lax.axis_index("s")
        wid = s * NC + c

        def zrow(i, carry):
            for j in range(D // 16):
                b0[i, pl.ds(j * 16, 16)] = jnp.zeros((16,), jnp.float32)
            return carry

        lax.fori_loop(0, 128, zrow, 0)
        rbase = s * RPS
        for k in range(RPS // 128):
            pltpu.sync_copy(b0, acc.at[pl.ds(rbase + k * 128, 128)])
        plsc.subcore_barrier()

        ib = wid * BPW
        for w in range(BPW // WIN):
            pltpu.sync_copy(src_hbm.at[pl.ds(ib + w * WIN, WIN)], src_v)
            pltpu.sync_copy(dst_hbm.at[pl.ds(ib + w * WIN, WIN)], dst_v)
            pltpu.async_copy(z_hbm.at[src_v.at[0]], b0, g0)

            def body(p, carry):
                jA = 2 * p
                jB = 2 * p + 1
                pltpu.make_async_copy(z_hbm.at[src_v.at[jA]], b0, g0).wait()
                pltpu.async_copy(z_hbm.at[src_v.at[jB]], b1, g1)
                pltpu.sync_copy(b0, acc.at[dst_v.at[jA]], add=True)
                pltpu.make_async_copy(z_hbm.at[src_v.at[jB]], b1, g1).wait()
                jC = jnp.minimum(jB + 1, WIN - 1)
                pltpu.async_copy(z_hbm.at[src_v.at[jC]], b0, g0)
                pltpu.sync_copy(b1, acc.at[dst_v.at[jB]], add=True)
                return carry

            lax.fori_loop(0, WIN // 2, body, 0)
            pltpu.make_async_copy(z_hbm.at[src_v.at[WIN - 1]], b0, g0).wait()
        plsc.subcore_barrier()
        pltpu.sync_copy(acc.at[pl.ds(rbase, RPS)],
                        out_hbm.at[c, pl.ds(rbase, RPS)])

    return prop(z, src_g, dst_g)


def _row_spec(last):
    return pl.BlockSpec((BLK, last), lambda i: (i, 0))


def _core_spec(core, last):
    return pl.BlockSpec((1, BLK, last), lambda i, _c=core: (_c, i, 0))


def _full_spec(a, b):
    return pl.BlockSpec((a, b), lambda i: (0, 0))


def _k1_body(dga, dgb, xr, dfr, bsr, h0_ref, z0_ref, dis_ref):
    deg = dga[0, :, 0:1] + dgb[0, :, 0:1]
    dis = jnp.where(deg > 0.0, lax.rsqrt(jnp.maximum(deg, 1.0)), 0.0)
    h0 = jnp.dot(xr[...], dfr[...], preferred_element_type=jnp.float32) + bsr[...]
    disb = dis * jnp.ones((1, D), jnp.float32)
    h0_ref[...] = h0
    dis_ref[...] = disb
    z0_ref[...] = disb * h0


def _tc_k1(degp, x16, diff16, basev):
    return pl.pallas_call(
        _k1_body,
        grid=(GRID,),
        in_specs=[
            _core_spec(0, D), _core_spec(1, D),
            _row_spec(16), _full_spec(16, D), _full_spec(1, D),
        ],
        out_specs=[_row_spec(D)] * 3,
        out_shape=[jax.ShapeDtypeStruct((N, D), jnp.float32)] * 3,
    )(degp, degp, x16, diff16, basev)


def _k2_body(gpa, gpb, hr, dr, w0r, w1r, acc_ref, u_ref):
    g = gpa[0] + gpb[0]
    d_ = dr[...]
    tx1 = -(d_ * g)
    acc_ref[...] = (jnp.dot(hr[...], w0r[...], preferred_element_type=jnp.float32)
                    + jnp.dot(tx1, w1r[...], preferred_element_type=jnp.float32))
    u_ref[...] = d_ * tx1


def _tc_k2(gp, h, disb, w0, w1):
    return pl.pallas_call(
        _k2_body,
        grid=(GRID,),
        in_specs=[
            _core_spec(0, D), _core_spec(1, D),
            _row_spec(D), _row_spec(D), _full_spec(D, D), _full_spec(D, D),
        ],
        out_specs=[_row_spec(D)] * 2,
        out_shape=[jax.ShapeDtypeStruct((N, D), jnp.float32)] * 2,
    )(gp, gp, h, disb, w0, w1)


def _k3_body(gpa, gpb, accr, hr, dr, w2r, br, hn_ref, zn_ref):
    g = gpa[0] + gpb[0]
    d_ = dr[...]
    tx2 = -2.0 * (d_ * g) - hr[...]
    hn = jnp.maximum(
        accr[...] + jnp.dot(tx2, w2r[...], preferred_element_type=jnp.float32)
        + br[...], 0.0)
    hn_ref[...] = hn
    zn_ref[...] = d_ * hn


def _tc_k3(gp, acc, h, disb, w2, b):
    return pl.pallas_call(
        _k3_body,
        grid=(GRID,),
        in_specs=[
            _core_spec(0, D), _core_spec(1, D),
            _row_spec(D), _row_spec(D), _row_spec(D),
            _full_spec(D, D), _full_spec(1, D),
        ],
        out_specs=[_row_spec(D)] * 2,
        out_shape=[jax.ShapeDtypeStruct((N, D), jnp.float32)] * 2,
    )(gp, gp, acc, h, disb, w2, b)


def _k56_body(gpa, gpb, accr, hr, dr, w2r, br, btr, gmr, bt2r,
              w1r, b1r, w2lr, b2r, out_ref, sacc, pool, cnt):
    i = pl.program_id(0)
    g = gpa[0] + gpb[0]
    d_ = dr[...]
    tx2 = -2.0 * (d_ * g) - hr[...]
    hn = jnp.maximum(
        accr[...] + jnp.dot(tx2, w2r[...], preferred_element_type=jnp.float32)
        + br[...], 0.0)
    ps = jnp.concatenate(
        [jnp.sum(hn, axis=0, keepdims=True),
         jnp.sum(hn * hn, axis=0, keepdims=True)], axis=0)
    mask = (btr[...] == lax.broadcasted_iota(jnp.int32, (1, NG), 1)
            ).astype(jnp.float32)
    pb = lax.dot_general(mask, hn, (((0,), (0,)), ((), ())),
                         preferred_element_type=jnp.float32)
    cb = lax.dot_general(mask, jnp.ones((BLK, 1), jnp.float32),
                         (((0,), (0,)), ((), ())),
                         preferred_element_type=jnp.float32)

    @pl.when(i == 0)
    def _():
        sacc[...] = ps
        pool[...] = pb
        cnt[...] = cb

    @pl.when(i > 0)
    def _():
        sacc[...] = sacc[...] + ps
        pool[...] = pool[...] + pb
        cnt[...] = cnt[...] + cb

    @pl.when(i == GRID - 1)
    def _():
        mean = sacc[0:1, :] * (1.0 / N)
        var = sacc[1:2, :] * (1.0 / N) - mean * mean
        inv = lax.rsqrt(var + 1e-5)
        pm = pool[...] / jnp.maximum(cnt[...], 1.0)
        y = (pm - mean) * inv * gmr[...] + bt2r[...]
        r = jnp.maximum(
            jnp.dot(y, w1r[...], preferred_element_type=jnp.float32)
            + b1r[...], 0.0)
        out_ref[...] = (jnp.dot(r, w2lr[...], preferred_element_type=jnp.float32)
                        + b2r[...])


def _tc_k56(gp, acc, h, disb, w2, b, bat2, gam, bet, w1, b1, w2l, b2):
    return pl.pallas_call(
        _k56_body,
        grid=(GRID,),
        in_specs=[
            _core_spec(0, D), _core_spec(1, D),
            _row_spec(D), _row_spec(D), _row_spec(D),
            _full_spec(D, D), _full_spec(1, D),
            _row_spec(1), _full_spec(1, D), _full_spec(1, D),
            _full_spec(D, 16), _full_spec(1, 16),
            _full_spec(16, 2), _full_spec(1, 2),
        ],
        out_specs=pl.BlockSpec((NG, 2), lambda i: (0, 0)),
        out_shape=jax.ShapeDtypeStruct((NG, 2), jnp.float32),
        scratch_shapes=[pltpu.VMEM((2, D), jnp.float32),
                        pltpu.VMEM((NG, D), jnp.float32),
                        pltpu.VMEM((NG, 1), jnp.float32)],
    )(gp, gp, acc, h, disb, w2, b, bat2, gam, bet, w1, b1, w2l, b2)


def kernel(x, edge_index, batch, atom_embs, conv1_W, conv1_b, conv3_W, conv3_b,
           bn_gamma, bn_beta, lin1_W, lin1_b, lin2_W, lin2_b):
    f32 = jnp.float32
    x16 = jnp.pad(x.astype(f32), ((0, 0), (0, 7)))
    diff16 = jnp.pad(
        jnp.stack([atom_embs[i][1] - atom_embs[i][0] for i in range(9)]),
        ((0, 7), (0, 0)))
    basev = functools.reduce(
        lambda a, b: a + b, [atom_embs[i][0] for i in range(9)]).reshape(1, D)

    src = edge_index[0]
    dst = edge_index[1]
    pad = E2 - E
    pidx = jnp.arange(pad, dtype=jnp.int32)
    dump = N + pidx % (NP - N)
    zpad = pidx % 128
    src_g = jnp.concatenate([src, zpad]).reshape(E2 // 128, 128)
    src_d = jnp.concatenate([src, dump]).reshape(E2 // 128, 128)
    dst_g = jnp.concatenate([dst, dump]).reshape(E2 // 128, 128)
    bat2 = batch.reshape(N, 1)

    degp = _sc_prop(jnp.ones((N, D), f32), src_g, src_d)
    h0, z0, disb = _tc_k1(degp, x16, diff16, basev)

    g1 = _sc_prop(z0, src_g, dst_g)
    acc1, u1 = _tc_k2(g1, h0, disb, conv1_W[0], conv1_W[1])
    g2 = _sc_prop(u1, src_g, dst_g)
    h1, z1 = _tc_k3(g2, acc1, h0, disb, conv1_W[2], conv1_b.reshape(1, D))

    g3 = _sc_prop(z1, src_g, dst_g)
    acc2, u2 = _tc_k2(g3, h1, disb, conv3_W[0], conv3_W[1])
    g4 = _sc_prop(u2, src_g, dst_g)

    return _tc_k56(g4, acc2, h1, disb, conv3_W[2], conv3_b.reshape(1, D),
                   bat2, bn_gamma.reshape(1, D), bn_beta.reshape(1, D),
                   lin1_W, lin1_b.reshape(1, 16), lin2_W, lin2_b.reshape(1, 2))


# speedup vs baseline: 1.1435x; 1.0057x over previous
import functools

import jax
import jax.numpy as jnp
from jax import lax
from jax.experimental import pallas as pl
from jax.experimental.pallas import tpu as pltpu
from jax.experimental.pallas import tpu_sc as plsc

N = 10000
D = 128
NG = 64
E = 320000
K = 3

NC = 2
NS = 16
NWK = NC * NS
BPW = 80
E2 = NWK * BPW * 128
NP = 10240
RPS = NP // NS
BLK = 2000
GRID = N // BLK
WIN = 16

_mesh = lambda: plsc.VectorSubcoreMesh(core_axis_name="c", subcore_axis_name="s")


def _sc_prop(z, src_g, dst_g):
    @functools.partial(
        pl.kernel,
        mesh=_mesh(),
        out_type=jax.ShapeDtypeStruct((NC, NP, D), jnp.float32),
        scratch_types=[
            pltpu.VMEM((WIN, 128), jnp.int32),
            pltpu.VMEM((WIN, 128), jnp.int32),
            pltpu.VMEM((128, D), jnp.float32),
            pltpu.VMEM((128, D), jnp.float32),
            pltpu.VMEM_SHARED((NP, D), jnp.float32),
            pltpu.SemaphoreType.DMA,
            pltpu.SemaphoreType.DMA,
            pltpu.SemaphoreType.DMA,
            pltpu.SemaphoreType.DMA,
        ],
    )
    def prop(z_hbm, src_hbm, dst_hbm, out_hbm, src_v, dst_v, b0, b1,
             acc, g0, g1, s0, s1):
        c = lax.axis_index("c")
        s = lax.axis_index("s")
        wid = s * NC + c

        def zrow(i, carry):
            for j in range(D // 16):
                b0[i, pl.ds(j * 16, 16)] = jnp.zeros((16,), jnp.float32)
            return carry

        lax.fori_loop(0, 128, zrow, 0)
        rbase = s * RPS
        for k in range(RPS // 128):
            pltpu.sync_copy(b0, acc.at[pl.ds(rbase + k * 128, 128)])
        plsc.subcore_barrier()

        ib = wid * BPW
        for w in range(BPW // WIN):
            pltpu.sync_copy(src_hbm.at[pl.ds(ib + w * WIN, WIN)], src_v)
            pltpu.sync_copy(dst_hbm.at[pl.ds(ib + w * WIN, WIN)], dst_v)
            pltpu.async_copy(z_hbm.at[src_v.at[0]], b0, g0)

            def body(p, carry):
                jA = 2 * p
                jB = 2 * p + 1
                pltpu.make_async_copy(z_hbm.at[src_v.at[jA]], b0, g0).wait()

                @pl.when(p > 0)
                def _():
                    pltpu.make_async_copy(
                        b1, acc.at[dst_v.at[jB - 2]], s1).wait()

                pltpu.async_copy(z_hbm.at[src_v.at[jB]], b1, g1)
                pltpu.async_copy(b0, acc.at[dst_v.at[jA]], s0, add=True)
                pltpu.make_async_copy(z_hbm.at[src_v.at[jB]], b1, g1).wait()
                pltpu.make_async_copy(b0, acc.at[dst_v.at[jA]], s0).wait()
                jC = jnp.minimum(jB + 1, WIN - 1)
                pltpu.async_copy(z_hbm.at[src_v.at[jC]], b0, g0)
                pltpu.async_copy(b1, acc.at[dst_v.at[jB]], s1, add=True)
                return carry

            lax.fori_loop(0, WIN // 2, body, 0)
            pltpu.make_async_copy(b1, acc.at[dst_v.at[WIN - 1]], s1).wait()
            pltpu.make_async_copy(z_hbm.at[src_v.at[WIN - 1]], b0, g0).wait()
        plsc.subcore_barrier()
        pltpu.sync_copy(acc.at[pl.ds(rbase, RPS)],
                        out_hbm.at[c, pl.ds(rbase, RPS)])

    return prop(z, src_g, dst_g)


def _row_spec(last):
    return pl.BlockSpec((BLK, last), lambda i: (i, 0))


def _core_spec(core, last):
    return pl.BlockSpec((1, BLK, last), lambda i, _c=core: (_c, i, 0))


def _full_spec(a, b):
    return pl.BlockSpec((a, b), lambda i: (0, 0))


def _k1_body(dga, dgb, xr, dfr, bsr, h0_ref, z0_ref, dis_ref):
    deg = dga[0, :, 0:1] + dgb[0, :, 0:1]
    dis = jnp.where(deg > 0.0, lax.rsqrt(jnp.maximum(deg, 1.0)), 0.0)
    h0 = jnp.dot(xr[...], dfr[...], preferred_element_type=jnp.float32) + bsr[...]
    disb = dis * jnp.ones((1, D), jnp.float32)
    h0_ref[...] = h0
    dis_ref[...] = disb
    z0_ref[...] = disb * h0


def _tc_k1(degp, x16, diff16, basev):
    return pl.pallas_call(
        _k1_body,
        grid=(GRID,),
        in_specs=[
            _core_spec(0, D), _core_spec(1, D),
            _row_spec(16), _full_spec(16, D), _full_spec(1, D),
        ],
        out_specs=[_row_spec(D)] * 3,
        out_shape=[jax.ShapeDtypeStruct((N, D), jnp.float32)] * 3,
    )(degp, degp, x16, diff16, basev)


def _k2_body(gpa, gpb, hr, dr, w0r, w1r, acc_ref, u_ref):
    g = gpa[0] + gpb[0]
    d_ = dr[...]
    tx1 = -(d_ * g)
    acc_ref[...] = (jnp.dot(hr[...], w0r[...], preferred_element_type=jnp.float32)
                    + jnp.dot(tx1, w1r[...], preferred_element_type=jnp.float32))
    u_ref[...] = d_ * tx1


def _tc_k2(gp, h, disb, w0, w1):
    return pl.pallas_call(
        _k2_body,
        grid=(GRID,),
        in_specs=[
            _core_spec(0, D), _core_spec(1, D),
            _row_spec(D), _row_spec(D), _full_spec(D, D), _full_spec(D, D),
        ],
        out_specs=[_row_spec(D)] * 2,
        out_shape=[jax.ShapeDtypeStruct((N, D), jnp.float32)] * 2,
    )(gp, gp, h, disb, w0, w1)


def _k3_body(gpa, gpb, accr, hr, dr, w2r, br, hn_ref, zn_ref):
    g = gpa[0] + gpb[0]
    d_ = dr[...]
    tx2 = -2.0 * (d_ * g) - hr[...]
    hn = jnp.maximum(
        accr[...] + jnp.dot(tx2, w2r[...], preferred_element_type=jnp.float32)
        + br[...], 0.0)
    hn_ref[...] = hn
    zn_ref[...] = d_ * hn


def _tc_k3(gp, acc, h, disb, w2, b):
    return pl.pallas_call(
        _k3_body,
        grid=(GRID,),
        in_specs=[
            _core_spec(0, D), _core_spec(1, D),
            _row_spec(D), _row_spec(D), _row_spec(D),
            _full_spec(D, D), _full_spec(1, D),
        ],
        out_specs=[_row_spec(D)] * 2,
        out_shape=[jax.ShapeDtypeStruct((N, D), jnp.float32)] * 2,
    )(gp, gp, acc, h, disb, w2, b)


def _k56_body(gpa, gpb, accr, hr, dr, w2r, br, btr, gmr, bt2r,
              w1r, b1r, w2lr, b2r, out_ref, sacc, pool, cnt):
    i = pl.program_id(0)
    g = gpa[0] + gpb[0]
    d_ = dr[...]
    tx2 = -2.0 * (d_ * g) - hr[...]
    hn = jnp.maximum(
        accr[...] + jnp.dot(tx2, w2r[...], preferred_element_type=jnp.float32)
        + br[...], 0.0)
    ps = jnp.concatenate(
        [jnp.sum(hn, axis=0, keepdims=True),
         jnp.sum(hn * hn, axis=0, keepdims=True)], axis=0)
    mask = (btr[...] == lax.broadcasted_iota(jnp.int32, (1, NG), 1)
            ).astype(jnp.float32)
    pb = lax.dot_general(mask, hn, (((0,), (0,)), ((), ())),
                         preferred_element_type=jnp.float32)
    cb = lax.dot_general(mask, jnp.ones((BLK, 1), jnp.float32),
                         (((0,), (0,)), ((), ())),
                         preferred_element_type=jnp.float32)

    @pl.when(i == 0)
    def _():
        sacc[...] = ps
        pool[...] = pb
        cnt[...] = cb

    @pl.when(i > 0)
    def _():
        sacc[...] = sacc[...] + ps
        pool[...] = pool[...] + pb
        cnt[...] = cnt[...] + cb

    @pl.when(i == GRID - 1)
    def _():
        mean = sacc[0:1, :] * (1.0 / N)
        var = sacc[1:2, :] * (1.0 / N) - mean * mean
        inv = lax.rsqrt(var + 1e-5)
        pm = pool[...] / jnp.maximum(cnt[...], 1.0)
        y = (pm - mean) * inv * gmr[...] + bt2r[...]
        r = jnp.maximum(
            jnp.dot(y, w1r[...], preferred_element_type=jnp.float32)
            + b1r[...], 0.0)
        out_ref[...] = (jnp.dot(r, w2lr[...], preferred_element_type=jnp.float32)
                        + b2r[...])


def _tc_k56(gp, acc, h, disb, w2, b, bat2, gam, bet, w1, b1, w2l, b2):
    return pl.pallas_call(
        _k56_body,
        grid=(GRID,),
        in_specs=[
            _core_spec(0, D), _core_spec(1, D),
            _row_spec(D), _row_spec(D), _row_spec(D),
            _full_spec(D, D), _full_spec(1, D),
            _row_spec(1), _full_spec(1, D), _full_spec(1, D),
            _full_spec(D, 16), _full_spec(1, 16),
            _full_spec(16, 2), _full_spec(1, 2),
        ],
        out_specs=pl.BlockSpec((NG, 2), lambda i: (0, 0)),
        out_shape=jax.ShapeDtypeStruct((NG, 2), jnp.float32),
        scratch_shapes=[pltpu.VMEM((2, D), jnp.float32),
                        pltpu.VMEM((NG, D), jnp.float32),
                        pltpu.VMEM((NG, 1), jnp.float32)],
    )(gp, gp, acc, h, disb, w2, b, bat2, gam, bet, w1, b1, w2l, b2)


def kernel(x, edge_index, batch, atom_embs, conv1_W, conv1_b, conv3_W, conv3_b,
           bn_gamma, bn_beta, lin1_W, lin1_b, lin2_W, lin2_b):
    f32 = jnp.float32
    x16 = jnp.pad(x.astype(f32), ((0, 0), (0, 7)))
    diff16 = jnp.pad(
        jnp.stack([atom_embs[i][1] - atom_embs[i][0] for i in range(9)]),
        ((0, 7), (0, 0)))
    basev = functools.reduce(
        lambda a, b: a + b, [atom_embs[i][0] for i in range(9)]).reshape(1, D)

    src = edge_index[0]
    dst = edge_index[1]
    pad = E2 - E
    pidx = jnp.arange(pad, dtype=jnp.int32)
    dump = N + pidx % (NP - N)
    zpad = pidx % 128
    src_g = jnp.concatenate([src, zpad]).reshape(E2 // 128, 128)
    src_d = jnp.concatenate([src, dump]).reshape(E2 // 128, 128)
    dst_g = jnp.concatenate([dst, dump]).reshape(E2 // 128, 128)
    bat2 = batch.reshape(N, 1)

    degp = _sc_prop(jnp.ones((N, D), f32), src_g, src_d)
    h0, z0, disb = _tc_k1(degp, x16, diff16, basev)

    g1 = _sc_prop(z0, src_g, dst_g)
    acc1, u1 = _tc_k2(g1, h0, disb, conv1_W[0], conv1_W[1])
    g2 = _sc_prop(u1, src_g, dst_g)
    h1, z1 = _tc_k3(g2, acc1, h0, disb, conv1_W[2], conv1_b.reshape(1, D))

    g3 = _sc_prop(z1, src_g, dst_g)
    acc2, u2 = _tc_k2(g3, h1, disb, conv3_W[0], conv3_W[1])
    g4 = _sc_prop(u2, src_g, dst_g)

    return _tc_k56(g4, acc2, h1, disb, conv3_W[2], conv3_b.reshape(1, D),
                   bat2, bn_gamma.reshape(1, D), bn_beta.reshape(1, D),
                   lin1_W, lin1_b.reshape(1, 16), lin2_W, lin2_b.reshape(1, 2))


# speedup vs baseline: 1.2022x; 1.0513x over previous
import functools

import jax
import jax.numpy as jnp
from jax import lax
from jax.experimental import pallas as pl
from jax.experimental.pallas import tpu as pltpu
from jax.experimental.pallas import tpu_sc as plsc

N = 10000
D = 128
NG = 64
E = 320000
K = 3

NC = 2
NS = 16
NWK = NC * NS
BPW = 80
E2 = NWK * BPW * 128
NP = 10240
RPS = NP // NS
BLK = 2000
GRID = N // BLK
WIN = 40

_mesh = lambda: plsc.VectorSubcoreMesh(core_axis_name="c", subcore_axis_name="s")


def _sc_prop(z, src_g, dst_g):
    @functools.partial(
        pl.kernel,
        mesh=_mesh(),
        out_type=jax.ShapeDtypeStruct((NC, NP, D), jnp.float32),
        scratch_types=[
            pltpu.VMEM((WIN, 128), jnp.int32),
            pltpu.VMEM((WIN, 128), jnp.int32),
            pltpu.VMEM((128, D), jnp.float32),
            pltpu.VMEM((128, D), jnp.float32),
            pltpu.VMEM_SHARED((NP, D), jnp.float32),
            pltpu.SemaphoreType.DMA,
            pltpu.SemaphoreType.DMA,
            pltpu.SemaphoreType.DMA,
            pltpu.SemaphoreType.DMA,
        ],
    )
    def prop(z_hbm, src_hbm, dst_hbm, out_hbm, src_v, dst_v, b0, b1,
             acc, g0, g1, s0, s1):
        c = lax.axis_index("c")
        s = lax.axis_index("s")
        wid = s * NC + c

        def zrow(i, carry):
            for j in range(D // 16):
                b0[i, pl.ds(j * 16, 16)] = jnp.zeros((16,), jnp.float32)
            return carry

        lax.fori_loop(0, 128, zrow, 0)
        rbase = s * RPS
        for k in range(RPS // 128):
            pltpu.sync_copy(b0, acc.at[pl.ds(rbase + k * 128, 128)])
        plsc.subcore_barrier()

        ib = wid * BPW
        for w in range(BPW // WIN):
            pltpu.sync_copy(src_hbm.at[pl.ds(ib + w * WIN, WIN)], src_v)
            pltpu.sync_copy(dst_hbm.at[pl.ds(ib + w * WIN, WIN)], dst_v)
            pltpu.async_copy(z_hbm.at[src_v.at[0]], b0, g0)

            def body(p, carry):
                jA = 2 * p
                jB = 2 * p + 1
                pltpu.make_async_copy(z_hbm.at[src_v.at[jA]], b0, g0).wait()

                @pl.when(p > 0)
                def _():
                    pltpu.make_async_copy(
                        b1, acc.at[dst_v.at[jB - 2]], s1).wait()

                pltpu.async_copy(z_hbm.at[src_v.at[jB]], b1, g1)
                pltpu.async_copy(b0, acc.at[dst_v.at[jA]], s0, add=True)
                pltpu.make_async_copy(z_hbm.at[src_v.at[jB]], b1, g1).wait()
                pltpu.make_async_copy(b0, acc.at[dst_v.at[jA]], s0).wait()
                jC = jnp.minimum(jB + 1, WIN - 1)
                pltpu.async_copy(z_hbm.at[src_v.at[jC]], b0, g0)
                pltpu.async_copy(b1, acc.at[dst_v.at[jB]], s1, add=True)
                return carry

            lax.fori_loop(0, WIN // 2, body, 0)
            pltpu.make_async_copy(b1, acc.at[dst_v.at[WIN - 1]], s1).wait()
            pltpu.make_async_copy(z_hbm.at[src_v.at[WIN - 1]], b0, g0).wait()
        plsc.subcore_barrier()
        pltpu.sync_copy(acc.at[pl.ds(rbase, RPS)],
                        out_hbm.at[c, pl.ds(rbase, RPS)])

    return prop(z, src_g, dst_g)


def _row_spec(last):
    return pl.BlockSpec((BLK, last), lambda i: (i, 0))


def _core_spec(core, last):
    return pl.BlockSpec((1, BLK, last), lambda i, _c=core: (_c, i, 0))


def _full_spec(a, b):
    return pl.BlockSpec((a, b), lambda i: (0, 0))


def _k1_body(dga, dgb, xr, dfr, bsr, h0_ref, z0_ref, dis_ref):
    deg = dga[0, :, 0:1] + dgb[0, :, 0:1]
    dis = jnp.where(deg > 0.0, lax.rsqrt(jnp.maximum(deg, 1.0)), 0.0)
    h0 = jnp.dot(xr[...], dfr[...], preferred_element_type=jnp.float32) + bsr[...]
    h0_ref[...] = h0
    dis_ref[...] = dis
    z0_ref[...] = dis * h0


def _tc_k1(degp, x16, diff16, basev):
    return pl.pallas_call(
        _k1_body,
        grid=(GRID,),
        in_specs=[
            _core_spec(0, D), _core_spec(1, D),
            _row_spec(16), _full_spec(16, D), _full_spec(1, D),
        ],
        out_specs=[_row_spec(D), _row_spec(D), _row_spec(1)],
        out_shape=[jax.ShapeDtypeStruct((N, D), jnp.float32),
                   jax.ShapeDtypeStruct((N, D), jnp.float32),
                   jax.ShapeDtypeStruct((N, 1), jnp.float32)],
    )(degp, degp, x16, diff16, basev)


def _k2_body(gpa, gpb, hr, dr, w0r, w1r, acc_ref, u_ref):
    g = gpa[0] + gpb[0]
    d_ = dr[...]
    tx1 = -(d_ * g)
    acc_ref[...] = (jnp.dot(hr[...], w0r[...], preferred_element_type=jnp.float32)
                    + jnp.dot(tx1, w1r[...], preferred_element_type=jnp.float32))
    u_ref[...] = d_ * tx1


def _tc_k2(gp, h, disb, w0, w1):
    return pl.pallas_call(
        _k2_body,
        grid=(GRID,),
        in_specs=[
            _core_spec(0, D), _core_spec(1, D),
            _row_spec(D), _row_spec(1), _full_spec(D, D), _full_spec(D, D),
        ],
        out_specs=[_row_spec(D)] * 2,
        out_shape=[jax.ShapeDtypeStruct((N, D), jnp.float32)] * 2,
    )(gp, gp, h, disb, w0, w1)


def _k3_body(gpa, gpb, accr, hr, dr, w2r, br, hn_ref, zn_ref):
    g = gpa[0] + gpb[0]
    d_ = dr[...]
    tx2 = -2.0 * (d_ * g) - hr[...]
    hn = jnp.maximum(
        accr[...] + jnp.dot(tx2, w2r[...], preferred_element_type=jnp.float32)
        + br[...], 0.0)
    hn_ref[...] = hn
    zn_ref[...] = d_ * hn


def _tc_k3(gp, acc, h, disb, w2, b):
    return pl.pallas_call(
        _k3_body,
        grid=(GRID,),
        in_specs=[
            _core_spec(0, D), _core_spec(1, D),
            _row_spec(D), _row_spec(D), _row_spec(1),
            _full_spec(D, D), _full_spec(1, D),
        ],
        out_specs=[_row_spec(D)] * 2,
        out_shape=[jax.ShapeDtypeStruct((N, D), jnp.float32)] * 2,
    )(gp, gp, acc, h, disb, w2, b)


def _k56_body(gpa, gpb, accr, hr, dr, w2r, br, btr, gmr, bt2r,
              w1r, b1r, w2lr, b2r, out_ref, sacc, pool, cnt):
    i = pl.program_id(0)
    g = gpa[0] + gpb[0]
    d_ = dr[...]
    tx2 = -2.0 * (d_ * g) - hr[...]
    hn = jnp.maximum(
        accr[...] + jnp.dot(tx2, w2r[...], preferred_element_type=jnp.float32)
        + br[...], 0.0)
    ps = jnp.concatenate(
        [jnp.sum(hn, axis=0, keepdims=True),
         jnp.sum(hn * hn, axis=0, keepdims=True)], axis=0)
    mask = (btr[...] == lax.broadcasted_iota(jnp.int32, (1, NG), 1)
            ).astype(jnp.float32)
    pb = lax.dot_general(mask, hn, (((0,), (0,)), ((), ())),
                         preferred_element_type=jnp.float32)
    cb = lax.dot_general(mask, jnp.ones((BLK, 1), jnp.float32),
                         (((0,), (0,)), ((), ())),
                         preferred_element_type=jnp.float32)

    @pl.when(i == 0)
    def _():
        sacc[...] = ps
        pool[...] = pb
        cnt[...] = cb

    @pl.when(i > 0)
    def _():
        sacc[...] = sacc[...] + ps
        pool[...] = pool[...] + pb
        cnt[...] = cnt[...] + cb

    @pl.when(i == GRID - 1)
    def _():
        mean = sacc[0:1, :] * (1.0 / N)
        var = sacc[1:2, :] * (1.0 / N) - mean * mean
        inv = lax.rsqrt(var + 1e-5)
        pm = pool[...] / jnp.maximum(cnt[...], 1.0)
        y = (pm - mean) * inv * gmr[...] + bt2r[...]
        r = jnp.maximum(
            jnp.dot(y, w1r[...], preferred_element_type=jnp.float32)
            + b1r[...], 0.0)
        out_ref[...] = (jnp.dot(r, w2lr[...], preferred_element_type=jnp.float32)
                        + b2r[...])


def _tc_k56(gp, acc, h, disb, w2, b, bat2, gam, bet, w1, b1, w2l, b2):
    return pl.pallas_call(
        _k56_body,
        grid=(GRID,),
        in_specs=[
            _core_spec(0, D), _core_spec(1, D),
            _row_spec(D), _row_spec(D), _row_spec(1),
            _full_spec(D, D), _full_spec(1, D),
            _row_spec(1), _full_spec(1, D), _full_spec(1, D),
            _full_spec(D, 16), _full_spec(1, 16),
            _full_spec(16, 2), _full_spec(1, 2),
        ],
        out_specs=pl.BlockSpec((NG, 2), lambda i: (0, 0)),
        out_shape=jax.ShapeDtypeStruct((NG, 2), jnp.float32),
        scratch_shapes=[pltpu.VMEM((2, D), jnp.float32),
                        pltpu.VMEM((NG, D), jnp.float32),
                        pltpu.VMEM((NG, 1), jnp.float32)],
    )(gp, gp, acc, h, disb, w2, b, bat2, gam, bet, w1, b1, w2l, b2)


def kernel(x, edge_index, batch, atom_embs, conv1_W, conv1_b, conv3_W, conv3_b,
           bn_gamma, bn_beta, lin1_W, lin1_b, lin2_W, lin2_b):
    f32 = jnp.float32
    x16 = jnp.pad(x.astype(f32), ((0, 0), (0, 7)))
    diff16 = jnp.pad(
        jnp.stack([atom_embs[i][1] - atom_embs[i][0] for i in range(9)]),
        ((0, 7), (0, 0)))
    basev = functools.reduce(
        lambda a, b: a + b, [atom_embs[i][0] for i in range(9)]).reshape(1, D)

    src = edge_index[0]
    dst = edge_index[1]
    pad = E2 - E
    pidx = jnp.arange(pad, dtype=jnp.int32)
    dump = N + pidx % (NP - N)
    zpad = pidx % 128
    src_g = jnp.concatenate([src, zpad]).reshape(E2 // 128, 128)
    src_d = jnp.concatenate([src, dump]).reshape(E2 // 128, 128)
    dst_g = jnp.concatenate([dst, dump]).reshape(E2 // 128, 128)
    bat2 = batch.reshape(N, 1)

    degp = _sc_prop(jnp.ones((N, D), f32), src_g, src_d)
    h0, z0, disb = _tc_k1(degp, x16, diff16, basev)

    g1 = _sc_prop(z0, src_g, dst_g)
    acc1, u1 = _tc_k2(g1, h0, disb, conv1_W[0], conv1_W[1])
    g2 = _sc_prop(u1, src_g, dst_g)
    h1, z1 = _tc_k3(g2, acc1, h0, disb, conv1_W[2], conv1_b.reshape(1, D))

    g3 = _sc_prop(z1, src_g, dst_g)
    acc2, u2 = _tc_k2(g3, h1, disb, conv3_W[0], conv3_W[1])
    g4 = _sc_prop(u2, src_g, dst_g)

    return _tc_k56(g4, acc2, h1, disb, conv3_W[2], conv3_b.reshape(1, D),
                   bat2, bn_gamma.reshape(1, D), bn_beta.reshape(1, D),
                   lin1_W, lin1_b.reshape(1, 16), lin2_W, lin2_b.reshape(1, 2))
